# baseline (speedup 1.0000x reference)
import functools

import jax
import jax.numpy as jnp
from jax import lax
from jax.experimental import pallas as pl
from jax.experimental.pallas import tpu as pltpu
from jax.experimental.pallas import tpu_sc as plsc

N = 1024
LANES = 16
ROW_W = 128
CHUNK_ROWS = 16
CHUNK = CHUNK_ROWS * ROW_W
NUM_SUBCORES = 16
NUM_CORES = 2
NBUF = 3
NUM_SAMPLES_EXPECTED = 2000000
TOTAL_ROWS = NUM_SAMPLES_EXPECTED // ROW_W
FULL_CHUNKS = NUM_SAMPLES_EXPECTED // CHUNK
CHUNKS_PER_SUBCORE = FULL_CHUNKS // NUM_SUBCORES
TAIL_ROWS = TOTAL_ROWS - FULL_CHUNKS * CHUNK_ROWS
REAL_BINS = N * N
ZCHUNK = 8192
ZERO_PER_SUBCORE = REAL_BINS // NUM_SUBCORES
OUT_PER_SUBCORE = REAL_BINS // NUM_SUBCORES


def _sc_body(cols_hbm, ab_hbm, cb_hbm, idx_v, b_v, ones_v, zbuf, tidx_v, tb_v,
             hist_sh, sem_in, sem_sc):
    core = lax.axis_index("c")
    sub = lax.axis_index("s")
    rowcol = core * 2

    for i in range(ROW_W // LANES):
        ones_v[pl.ds(i * LANES, LANES)] = jnp.ones((LANES,), jnp.float32)

    zero16 = jnp.zeros((LANES,), jnp.float32)

    def zfill(i, carry):
        zbuf[pl.ds(i * LANES, LANES)] = zero16
        return carry

    lax.fori_loop(0, ZCHUNK // LANES, zfill, 0)
    zbase = sub * ZERO_PER_SUBCORE
    for k in range(ZERO_PER_SUBCORE // ZCHUNK):
        pltpu.sync_copy(zbuf, hist_sh.at[pl.ds(zbase + k * ZCHUNK, ZCHUNK)])
    plsc.subcore_barrier()

    def start_in(chunk_id, p):
        rb = chunk_id * CHUNK_ROWS
        pltpu.async_copy(cols_hbm.at[core, pl.ds(rb, CHUNK_ROWS)],
                         idx_v.at[p], sem_in.at[p])

    def wait_in(p):
        pltpu.make_async_copy(cols_hbm.at[0, pl.ds(0, CHUNK_ROWS)],
                              idx_v.at[p], sem_in.at[p]).wait()

    def fire_sc(p, rows=CHUNK_ROWS):
        for r in range(rows):
            pltpu.async_copy(ones_v, hist_sh.at[idx_v.at[p].at[r]],
                             sem_sc.at[p], add=True)

    def drain_sc(p, rows=CHUNK_ROWS):
        for r in range(rows):
            pltpu.make_async_copy(ones_v, hist_sh.at[idx_v.at[p].at[r]],
                                  sem_sc.at[p]).wait()

    chunk0 = sub * CHUNKS_PER_SUBCORE

    start_in(chunk0, 0)

    def phase(g, p):
        @pl.when(g >= 2)
        def _():
            drain_sc((p + 1) % NBUF)

        @pl.when(g + 1 < CHUNKS_PER_SUBCORE)
        def _():
            start_in(chunk0 + g + 1, (p + 1) % NBUF)

        wait_in(p)
        fire_sc(p)

    def iter_body(it, carry):
        for p in range(NBUF):
            phase(it * NBUF + p, p)
        return carry

    lax.fori_loop(0, CHUNKS_PER_SUBCORE // NBUF, iter_body, 0)
    phase(jnp.int32(CHUNKS_PER_SUBCORE - 1), (CHUNKS_PER_SUBCORE - 1) % NBUF)
    drain_sc((CHUNKS_PER_SUBCORE - 2) % NBUF)
    drain_sc((CHUNKS_PER_SUBCORE - 1) % NBUF)

    @pl.when(sub == NUM_SUBCORES - 1)
    def _():
        tb = FULL_CHUNKS * CHUNK_ROWS
        pltpu.sync_copy(cols_hbm.at[core, pl.ds(tb, TAIL_ROWS)], tidx_v)
        for r in range(TAIL_ROWS):
            pltpu.sync_copy(ones_v, hist_sh.at[tidx_v.at[r]], add=True)

    plsc.subcore_barrier()

    obase = sub * OUT_PER_SUBCORE

    @pl.when(core == 0)
    def _():
        pltpu.sync_copy(hist_sh.at[pl.ds(obase, OUT_PER_SUBCORE)],
                        ab_hbm.at[pl.ds(obase, OUT_PER_SUBCORE)])

    @pl.when(core != 0)
    def _():
        pltpu.sync_copy(hist_sh.at[pl.ds(obase, OUT_PER_SUBCORE)],
                        cb_hbm.at[pl.ds(obase, OUT_PER_SUBCORE)])


_sc_hist = functools.partial(
    pl.kernel,
    out_type=[jax.ShapeDtypeStruct((REAL_BINS,), jnp.float32),
              jax.ShapeDtypeStruct((REAL_BINS,), jnp.float32)],
    mesh=plsc.VectorSubcoreMesh(core_axis_name="c", subcore_axis_name="s"),
    scratch_types=[
        pltpu.VMEM((NBUF, CHUNK_ROWS, ROW_W), jnp.int32),
        pltpu.VMEM((NBUF, CHUNK_ROWS, ROW_W), jnp.int32),
        pltpu.VMEM((ROW_W,), jnp.float32),
        pltpu.VMEM((ZCHUNK,), jnp.float32),
        pltpu.VMEM((TAIL_ROWS, ROW_W), jnp.int32),
        pltpu.VMEM((TAIL_ROWS, ROW_W), jnp.int32),
        pltpu.VMEM_SHARED((REAL_BINS,), jnp.float32),
        pltpu.SemaphoreType.DMA((NBUF,)),
        pltpu.SemaphoreType.DMA((NBUF,)),
    ],
)(_sc_body)


def _tc_norm_body(ab_ref, cb_ref, o_ref, *, num_samples):
    ab = ab_ref[...]
    cb = cb_ref[...]
    pib = (jnp.sum(ab, axis=0) * (1.0 / num_samples)).reshape(1, N)
    abn = ab / jnp.maximum(jnp.sum(ab, axis=(1, 2), keepdims=True), 1.0)
    cbn = cb / jnp.maximum(jnp.sum(cb, axis=(1, 2), keepdims=True), 1.0)
    o_ref[...] = jnp.concatenate(
        [pib, abn.reshape(N, N), cbn.reshape(N, N)], axis=0)


def kernel(inputs):
    num_samples = inputs.shape[0]
    cols = jnp.stack([inputs[:, 0] * N + inputs[:, 1],
                      inputs[:, 2] * N + inputs[:, 1]]
                     ).reshape(2, TOTAL_ROWS, ROW_W)

    ab1, cb1 = _sc_hist(cols)
    ab3 = ab1.reshape(N, 8, ROW_W)
    cb3 = cb1.reshape(N, 8, ROW_W)

    out = pl.pallas_call(
        functools.partial(_tc_norm_body, num_samples=float(num_samples)),
        out_shape=jax.ShapeDtypeStruct((2 * N + 1, N), jnp.float32),
    )(ab3, cb3)
    return out


# speedup vs baseline: 1.9612x; 1.9612x over previous
import functools

import jax
import jax.numpy as jnp
from jax import lax
from jax.experimental import pallas as pl
from jax.experimental.pallas import tpu as pltpu
from jax.experimental.pallas import tpu_sc as plsc

N = 1024
LANES = 16
ROW_W = 128
CHUNK_ROWS = 16
CHUNK = CHUNK_ROWS * ROW_W
NUM_SUBCORES = 16
NUM_CORES = 2
NBUF = 3
NUM_SAMPLES_EXPECTED = 2000000
TOTAL_ROWS = NUM_SAMPLES_EXPECTED // ROW_W
FULL_CHUNKS = NUM_SAMPLES_EXPECTED // CHUNK
CHUNKS_PER_SUBCORE = FULL_CHUNKS // NUM_SUBCORES
TAIL = NUM_SAMPLES_EXPECTED - FULL_CHUNKS * CHUNK
REAL_BINS = N * N
ZCHUNK = 8192
ZERO_PER_SUBCORE = REAL_BINS // NUM_SUBCORES
OUT_PER_SUBCORE = REAL_BINS // NUM_SUBCORES


def _sc_body(cols_hbm, ab_hbm, cb_hbm, idx_v, b_v, ones_v, zbuf, tidx_v, tb_v,
             hist_sh, sem_in, sem_sc):
    core = lax.axis_index("c")
    sub = lax.axis_index("s")
    rowcol = core * 2

    for i in range(CHUNK // LANES):
        ones_v[0, pl.ds(i * LANES, LANES)] = jnp.ones((LANES,), jnp.float32)

    zero16 = jnp.zeros((LANES,), jnp.float32)

    def zfill(i, carry):
        zbuf[pl.ds(i * LANES, LANES)] = zero16
        return carry

    lax.fori_loop(0, ZCHUNK // LANES, zfill, 0)
    zbase = sub * ZERO_PER_SUBCORE
    for k in range(ZERO_PER_SUBCORE // ZCHUNK):
        pltpu.sync_copy(zbuf, hist_sh.at[pl.ds(zbase + k * ZCHUNK, ZCHUNK)])
    plsc.subcore_barrier()

    def start_in(chunk_id, p):
        sb = chunk_id * CHUNK
        pltpu.async_copy(cols_hbm.at[rowcol, pl.ds(sb, CHUNK)],
                         idx_v.at[p].at[0], sem_in.at[p])
        pltpu.async_copy(cols_hbm.at[1, pl.ds(sb, CHUNK)],
                         b_v.at[p].at[0], sem_in.at[p])

    def wait_in(p):
        pltpu.make_async_copy(cols_hbm.at[0, pl.ds(0, CHUNK)],
                              idx_v.at[p].at[0], sem_in.at[p]).wait()
        pltpu.make_async_copy(cols_hbm.at[1, pl.ds(0, CHUNK)],
                              b_v.at[p].at[0], sem_in.at[p]).wait()

    def fire_sc(p):
        pltpu.async_copy(ones_v.at[0], hist_sh.at[idx_v.at[p].at[0]],
                         sem_sc.at[p], add=True)

    def drain_sc(p):
        pltpu.make_async_copy(ones_v.at[0], hist_sh.at[idx_v.at[p].at[0]],
                              sem_sc.at[p]).wait()

    chunk0 = sub * CHUNKS_PER_SUBCORE

    start_in(chunk0, 0)

    def phase(g, p):
        @pl.when(g >= 2)
        def _():
            drain_sc((p + 1) % NBUF)

        @pl.when(g + 1 < CHUNKS_PER_SUBCORE)
        def _():
            start_in(chunk0 + g + 1, (p + 1) % NBUF)

        wait_in(p)
        for j in range(CHUNK // LANES):
            sl = pl.ds(j * LANES, LANES)
            idx_v[p, 0, sl] = idx_v[p, 0, sl] * N + b_v[p, 0, sl]
        fire_sc(p)

    def iter_body(it, carry):
        for p in range(NBUF):
            phase(it * NBUF + p, p)
        return carry

    lax.fori_loop(0, CHUNKS_PER_SUBCORE // NBUF, iter_body, 0)
    phase(jnp.int32(CHUNKS_PER_SUBCORE - 1), (CHUNKS_PER_SUBCORE - 1) % NBUF)
    drain_sc((CHUNKS_PER_SUBCORE - 2) % NBUF)
    drain_sc((CHUNKS_PER_SUBCORE - 1) % NBUF)

    @pl.when(sub == NUM_SUBCORES - 1)
    def _():
        tb = FULL_CHUNKS * CHUNK
        pltpu.sync_copy(cols_hbm.at[rowcol, pl.ds(tb, TAIL)], tidx_v.at[0])
        pltpu.sync_copy(cols_hbm.at[1, pl.ds(tb, TAIL)], tb_v.at[0])
        for j in range(TAIL // LANES):
            sl = pl.ds(j * LANES, LANES)
            tidx_v[0, sl] = tidx_v[0, sl] * N + tb_v[0, sl]
        pltpu.sync_copy(ones_v.at[0].at[pl.ds(0, TAIL)],
                        hist_sh.at[tidx_v.at[0]], add=True)

    plsc.subcore_barrier()

    obase = sub * OUT_PER_SUBCORE

    @pl.when(core == 0)
    def _():
        pltpu.sync_copy(hist_sh.at[pl.ds(obase, OUT_PER_SUBCORE)],
                        ab_hbm.at[pl.ds(obase, OUT_PER_SUBCORE)])

    @pl.when(core != 0)
    def _():
        pltpu.sync_copy(hist_sh.at[pl.ds(obase, OUT_PER_SUBCORE)],
                        cb_hbm.at[pl.ds(obase, OUT_PER_SUBCORE)])


_sc_hist = functools.partial(
    pl.kernel,
    out_type=[jax.ShapeDtypeStruct((REAL_BINS,), jnp.float32),
              jax.ShapeDtypeStruct((REAL_BINS,), jnp.float32)],
    mesh=plsc.VectorSubcoreMesh(core_axis_name="c", subcore_axis_name="s"),
    scratch_types=[
        pltpu.VMEM((NBUF, 1, CHUNK), jnp.int32),
        pltpu.VMEM((NBUF, 1, CHUNK), jnp.int32),
        pltpu.VMEM((1, CHUNK), jnp.float32),
        pltpu.VMEM((ZCHUNK,), jnp.float32),
        pltpu.VMEM((1, TAIL), jnp.int32),
        pltpu.VMEM((1, TAIL), jnp.int32),
        pltpu.VMEM_SHARED((REAL_BINS,), jnp.float32),
        pltpu.SemaphoreType.DMA((NBUF,)),
        pltpu.SemaphoreType.DMA((NBUF,)),
    ],
)(_sc_body)


def _tc_norm_body(ab_ref, cb_ref, o_ref, *, num_samples):
    ab = ab_ref[...]
    cb = cb_ref[...]
    pib = (jnp.sum(ab, axis=0) * (1.0 / num_samples)).reshape(1, N)
    abn = ab / jnp.maximum(jnp.sum(ab, axis=(1, 2), keepdims=True), 1.0)
    cbn = cb / jnp.maximum(jnp.sum(cb, axis=(1, 2), keepdims=True), 1.0)
    o_ref[...] = jnp.concatenate(
        [pib, abn.reshape(N, N), cbn.reshape(N, N)], axis=0)


def kernel(inputs):
    num_samples = inputs.shape[0]
    cols = inputs.T

    ab1, cb1 = _sc_hist(cols)
    ab3 = ab1.reshape(N, 8, ROW_W)
    cb3 = cb1.reshape(N, 8, ROW_W)

    out = pl.pallas_call(
        functools.partial(_tc_norm_body, num_samples=float(num_samples)),
        out_shape=jax.ShapeDtypeStruct((2 * N + 1, N), jnp.float32),
    )(ab3, cb3)
    return out
